# baseline (speedup 1.0000x reference)
import jax
import jax.numpy as jnp
from jax import lax
from jax.experimental import pallas as pl
from jax.experimental.pallas import tpu as pltpu

D_MODEL = 4096
N_EXP = 64
K = 8
CHUNK_T = 512
NBUF = 4
N_TOK = 16384


def _router_body(x_hbm, w_ref, b_ref, gates_ref, idx_ref, buf, sems):
    n_chunks = N_TOK // CHUNK_T
    w = w_ref[...]
    bias = b_ref[...]
    fiota = lax.broadcasted_iota(
        jnp.int32, (CHUNK_T, N_EXP), 1).astype(jnp.float32)
    neg_inf = jnp.float32(float("-inf"))
    big = jnp.float32(N_EXP)

    def start(i, slot):
        pltpu.make_async_copy(
            x_hbm.at[pl.ds(i * CHUNK_T, CHUNK_T), :],
            buf.at[slot],
            sems.at[slot],
        ).start()

    def wait(i, slot):
        pltpu.make_async_copy(
            x_hbm.at[pl.ds(i * CHUNK_T, CHUNK_T), :],
            buf.at[slot],
            sems.at[slot],
        ).wait()

    for s in range(NBUF):
        start(s, s)

    def loop(i, carry):
        slot = lax.rem(i, NBUF)
        wait(i, slot)
        x_blk = buf[slot]
        logits = lax.dot_general(
            x_blk, w, (((1,), (1,)), ((), ())),
            preferred_element_type=jnp.float32,
        ) + bias

        nxt = i + NBUF

        @pl.when(nxt < n_chunks)
        def _():
            start(nxt, slot)

        vals = []
        idxs = []
        l = logits
        for _ in range(K):
            m = jnp.max(l, axis=1, keepdims=True)
            cand = jnp.where(l == m, fiota, big)
            a = jnp.min(cand, axis=1, keepdims=True)
            vals.append(m)
            idxs.append(a)
            l = jnp.where(fiota == a, neg_inf, l)

        v = jnp.concatenate(vals, axis=1)
        e = jnp.exp(v - vals[0])
        g = e / jnp.sum(e, axis=1, keepdims=True)
        base = i * CHUNK_T
        gates_ref[pl.ds(base, CHUNK_T), :] = g
        idx_ref[pl.ds(base, CHUNK_T), :] = jnp.concatenate(
            idxs, axis=1).astype(jnp.int32)
        return carry

    lax.fori_loop(0, n_chunks, loop, 0, unroll=False)


@jax.jit
def kernel(x, W, b):
    B, S, D = x.shape
    T = B * S
    xf = x.reshape(T, D)
    b2 = b.reshape(1, N_EXP)
    gates, idx = pl.pallas_call(
        _router_body,
        in_specs=[
            pl.BlockSpec(memory_space=pl.ANY),
            pl.BlockSpec(memory_space=pltpu.VMEM),
            pl.BlockSpec(memory_space=pltpu.VMEM),
        ],
        out_specs=[
            pl.BlockSpec(memory_space=pltpu.VMEM),
            pl.BlockSpec(memory_space=pltpu.VMEM),
        ],
        out_shape=[
            jax.ShapeDtypeStruct((T, K), jnp.float32),
            jax.ShapeDtypeStruct((T, K), jnp.int32),
        ],
        scratch_shapes=[
            pltpu.VMEM((NBUF, CHUNK_T, D_MODEL), jnp.float32),
            pltpu.SemaphoreType.DMA((NBUF,)),
        ],
    )(xf, W, b2)
    return gates.reshape(B, S, K), idx.reshape(B, S, K)


# speedup vs baseline: 1.0170x; 1.0170x over previous
import jax
import jax.numpy as jnp
from jax import lax
from jax.experimental import pallas as pl
from jax.experimental.pallas import tpu as pltpu

D_MODEL = 4096
N_EXP = 64
K = 8
CHUNK_T = 512
NBUF = 4
N_TOK = 16384


def _router_body(x_hbm, w_ref, b_ref, gates_ref, idx_ref, buf, sems):
    n_chunks = N_TOK // CHUNK_T
    w = w_ref[...]
    bias = b_ref[...]
    fiota = lax.broadcasted_iota(
        jnp.int32, (N_EXP, CHUNK_T), 0).astype(jnp.float32)
    neg_inf = jnp.float32(float("-inf"))
    big = jnp.float32(N_EXP)

    def start(i, slot):
        pltpu.make_async_copy(
            x_hbm.at[pl.ds(i * CHUNK_T, CHUNK_T), :],
            buf.at[slot],
            sems.at[slot],
        ).start()

    def wait(i, slot):
        pltpu.make_async_copy(
            x_hbm.at[pl.ds(i * CHUNK_T, CHUNK_T), :],
            buf.at[slot],
            sems.at[slot],
        ).wait()

    for s in range(NBUF):
        start(s, s)

    def loop(i, carry):
        slot = lax.rem(i, NBUF)
        wait(i, slot)
        x_blk = buf[slot]
        logits = lax.dot_general(
            w, x_blk, (((1,), (1,)), ((), ())),
            preferred_element_type=jnp.float32,
        ) + bias

        nxt = i + NBUF

        @pl.when(nxt < n_chunks)
        def _():
            start(nxt, slot)

        vals = []
        idxs = []
        l = logits
        for _ in range(K):
            m = jnp.max(l, axis=0, keepdims=True)
            cand = jnp.where(l == m, fiota, big)
            a = jnp.min(cand, axis=0, keepdims=True)
            vals.append(m)
            idxs.append(a)
            l = jnp.where(fiota == a, neg_inf, l)

        v = jnp.concatenate(vals, axis=0)
        e = jnp.exp(v - vals[0])
        g = e / jnp.sum(e, axis=0, keepdims=True)
        gi = jnp.concatenate(idxs, axis=0).astype(jnp.int32)
        base = i * CHUNK_T
        gates_ref[pl.ds(base, CHUNK_T), :] = g.T
        idx_ref[pl.ds(base, CHUNK_T), :] = gi.T
        return carry

    lax.fori_loop(0, n_chunks, loop, 0, unroll=False)


@jax.jit
def kernel(x, W, b):
    B, S, D = x.shape
    T = B * S
    xf = x.reshape(T, D)
    b2 = b.reshape(N_EXP, 1)
    gates, idx = pl.pallas_call(
        _router_body,
        in_specs=[
            pl.BlockSpec(memory_space=pl.ANY),
            pl.BlockSpec(memory_space=pltpu.VMEM),
            pl.BlockSpec(memory_space=pltpu.VMEM),
        ],
        out_specs=[
            pl.BlockSpec(memory_space=pltpu.VMEM),
            pl.BlockSpec(memory_space=pltpu.VMEM),
        ],
        out_shape=[
            jax.ShapeDtypeStruct((T, K), jnp.float32),
            jax.ShapeDtypeStruct((T, K), jnp.int32),
        ],
        scratch_shapes=[
            pltpu.VMEM((NBUF, CHUNK_T, D_MODEL), jnp.float32),
            pltpu.SemaphoreType.DMA((NBUF,)),
        ],
    )(xf, W, b2)
    return gates.reshape(B, S, K), idx.reshape(B, S, K)
